# Initial kernel scaffold; baseline (speedup 1.0000x reference)
import jax
import jax.numpy as jnp
from jax.experimental import pallas as pl


def kernel(input_ids, word_embeddings, token_type_embeddings, position_embeddings):
    raise NotImplementedError("write your pallas kernel here")


# speedup vs baseline: 2.5960x; 2.5960x over previous
import functools

import jax
import jax.numpy as jnp
from jax import lax
from jax.experimental import pallas as pl
from jax.experimental.pallas import tpu as pltpu
from jax.experimental.pallas import tpu_sc as plsc

B = 1024
S = 200
HIDDEN = 128
LANES = 16
NH = HIDDEN // LANES

NUM_CORES = 2
NUM_SUBCORES = 16
NW = NUM_CORES * NUM_SUBCORES

TOTAL = B * S
CHUNK = 128
ROWS_PER_W = TOTAL // NW
CHUNKS_PER_W = ROWS_PER_W // CHUNK


def _sc_body(ids_hbm, word_hbm, tt_hbm, pos_hbm, out_hbm,
             idx_v, bias_v, tt_v, rows_v, gsem, ssem):
    wid = lax.axis_index("s") * NUM_CORES + lax.axis_index("c")

    pltpu.sync_copy(ids_hbm.at[pl.ds(wid * ROWS_PER_W, ROWS_PER_W)], idx_v)

    pltpu.sync_copy(pos_hbm.at[pl.ds(0, S)], bias_v)
    pltpu.sync_copy(tt_hbm, tt_v)
    tt_vecs = [tt_v[0, pl.ds(h * LANES, LANES)] for h in range(NH)]

    def bias_row(s, carry):
        for h in range(NH):
            plsc.addupdate(bias_v.at[s, pl.ds(h * LANES, LANES)], tt_vecs[h])
        return carry

    lax.fori_loop(0, S, bias_row, None)

    out_base = wid * ROWS_PER_W

    def chunk_body(c, carry):
        pltpu.async_copy(word_hbm.at[idx_v.at[pl.ds(c * CHUNK, CHUNK)]],
                         rows_v, gsem).wait()

        s0 = lax.rem(c * CHUNK, S)

        def add_row(i, s):
            for h in range(NH):
                sl = pl.ds(h * LANES, LANES)
                plsc.addupdate(rows_v.at[i, sl], bias_v[s, sl])
            return lax.select(s == S - 1, 0, s + 1)

        lax.fori_loop(0, CHUNK, add_row, s0)

        pltpu.async_copy(rows_v, out_hbm.at[pl.ds(out_base + c * CHUNK, CHUNK)],
                         ssem).wait()
        return carry

    lax.fori_loop(0, CHUNKS_PER_W, chunk_body, None)


@jax.jit
def kernel(input_ids, word_embeddings, token_type_embeddings, position_embeddings):
    ids_flat = input_ids.astype(jnp.int32).reshape(TOTAL)

    mesh = plsc.VectorSubcoreMesh(
        core_axis_name="c", subcore_axis_name="s",
        num_cores=NUM_CORES, num_subcores=NUM_SUBCORES)

    run = pl.kernel(
        _sc_body,
        out_type=jax.ShapeDtypeStruct((TOTAL, HIDDEN), jnp.float32),
        mesh=mesh,
        scratch_types=[
            pltpu.VMEM((ROWS_PER_W,), jnp.int32),
            pltpu.VMEM((S, HIDDEN), jnp.float32),
            pltpu.VMEM((2, HIDDEN), jnp.float32),
            pltpu.VMEM((CHUNK, HIDDEN), jnp.float32),
            pltpu.SemaphoreType.DMA,
            pltpu.SemaphoreType.DMA,
        ],
    )
    out = run(ids_flat, word_embeddings, token_type_embeddings,
              position_embeddings)
    return out.reshape(B, S, HIDDEN)


# speedup vs baseline: 3.8486x; 1.4825x over previous
import jax
import jax.numpy as jnp
from jax import lax
from jax.experimental import pallas as pl
from jax.experimental.pallas import tpu as pltpu
from jax.experimental.pallas import tpu_sc as plsc

B = 1024
S = 200
HIDDEN = 128
LANES = 16
NH = HIDDEN // LANES

NUM_CORES = 2
NUM_SUBCORES = 16
NW = NUM_CORES * NUM_SUBCORES

TOTAL = B * S
CHUNK = 128
ROWS_PER_W = TOTAL // NW
CHUNKS_PER_W = ROWS_PER_W // CHUNK

NBUF = 5
LOOKAHEAD = 3
ROUNDS = CHUNKS_PER_W // NBUF


def _sc_body(ids_hbm, word_hbm, tt_hbm, pos_hbm, out_hbm,
             idx_v, bias_v, tt_v, rows_v, gsem, ssem):
    wid = lax.axis_index("s") * NUM_CORES + lax.axis_index("c")

    pltpu.sync_copy(ids_hbm.at[pl.ds(wid * ROWS_PER_W, ROWS_PER_W)], idx_v)

    pltpu.sync_copy(pos_hbm.at[pl.ds(0, S)], bias_v)
    pltpu.sync_copy(tt_hbm, tt_v)
    tt_vecs = [tt_v[0, pl.ds(h * LANES, LANES)] for h in range(NH)]

    def bias_row(s, carry):
        for h in range(NH):
            plsc.addupdate(bias_v.at[s, pl.ds(h * LANES, LANES)], tt_vecs[h])
        return carry

    lax.fori_loop(0, S, bias_row, None)

    out_base = wid * ROWS_PER_W

    def gather_start(c, b):
        pltpu.async_copy(word_hbm.at[idx_v.at[pl.ds(c * CHUNK, CHUNK)]],
                         rows_v.at[b], gsem.at[b])

    def gather_wait(c, b):
        pltpu.make_async_copy(word_hbm.at[idx_v.at[pl.ds(c * CHUNK, CHUNK)]],
                              rows_v.at[b], gsem.at[b]).wait()

    def scatter_start(c, b):
        pltpu.async_copy(rows_v.at[b],
                         out_hbm.at[pl.ds(out_base + c * CHUNK, CHUNK)],
                         ssem.at[b])

    def scatter_wait(c, b):
        pltpu.make_async_copy(rows_v.at[b],
                              out_hbm.at[pl.ds(out_base + c * CHUNK, CHUNK)],
                              ssem.at[b]).wait()

    def add_bias(c, b):
        s0 = lax.rem(c * CHUNK, S)

        def add_row(i, s):
            for h in range(NH):
                sl = pl.ds(h * LANES, LANES)
                plsc.addupdate(rows_v.at[b, i, sl], bias_v[s, sl])
            return lax.select(s == S - 1, 0, s + 1)

        lax.fori_loop(0, CHUNK, add_row, s0)

    def visit(c, b, swait, gstart):
        bn = (b + LOOKAHEAD) % NBUF
        if swait:
            scatter_wait(c - (NBUF - LOOKAHEAD), bn)
        if gstart:
            gather_start(c + LOOKAHEAD, bn)
        gather_wait(c, b)
        add_bias(c, b)
        scatter_start(c, b)

    for b in range(LOOKAHEAD):
        gather_start(b, b)

    for b in range(NBUF):
        visit(b, b, swait=(b >= NBUF - LOOKAHEAD), gstart=True)

    def round_body(r, carry):
        for b in range(NBUF):
            visit(r * NBUF + b, b, swait=True, gstart=True)
        return carry

    lax.fori_loop(1, ROUNDS - 1, round_body, None)

    c0 = (ROUNDS - 1) * NBUF
    for b in range(NBUF):
        last = c0 + b + LOOKAHEAD < CHUNKS_PER_W
        visit(c0 + b, b, swait=last, gstart=last)

    for b in range(NBUF):
        scatter_wait(c0 + b, b)


@jax.jit
def kernel(input_ids, word_embeddings, token_type_embeddings, position_embeddings):
    ids_flat = input_ids.astype(jnp.int32).reshape(TOTAL)

    mesh = plsc.VectorSubcoreMesh(
        core_axis_name="c", subcore_axis_name="s",
        num_cores=NUM_CORES, num_subcores=NUM_SUBCORES)

    run = pl.kernel(
        _sc_body,
        out_type=jax.ShapeDtypeStruct((TOTAL, HIDDEN), jnp.float32),
        mesh=mesh,
        scratch_types=[
            pltpu.VMEM((ROWS_PER_W,), jnp.int32),
            pltpu.VMEM((S, HIDDEN), jnp.float32),
            pltpu.VMEM((2, HIDDEN), jnp.float32),
            pltpu.VMEM((NBUF, CHUNK, HIDDEN), jnp.float32),
            pltpu.SemaphoreType.DMA((NBUF,)),
            pltpu.SemaphoreType.DMA((NBUF,)),
        ],
    )
    out = run(ids_flat, word_embeddings, token_type_embeddings,
              position_embeddings)
    return out.reshape(B, S, HIDDEN)


# speedup vs baseline: 7.4948x; 1.9474x over previous
import jax
import jax.numpy as jnp
from jax import lax
from jax.experimental import pallas as pl
from jax.experimental.pallas import tpu as pltpu
from jax.experimental.pallas import tpu_sc as plsc

B = 1024
S = 200
HIDDEN = 128
LANES = 16
NH = HIDDEN // LANES

NUM_CORES = 2
NUM_SUBCORES = 16
NW = NUM_CORES * NUM_SUBCORES

TOTAL = B * S
CHUNK = 128
ROWS_PER_W = TOTAL // NW
CHUNKS_PER_W = ROWS_PER_W // CHUNK

SEXT = S + CHUNK - 8

NBUF = 5
LOOKAHEAD = 3
ROUNDS = CHUNKS_PER_W // NBUF


def _sc_body(ids_hbm, word_hbm, tt_hbm, pos_hbm, out_hbm,
             idx_v, bias_v, tt_v, rows_v, gsem, ssem):
    wid = lax.axis_index("s") * NUM_CORES + lax.axis_index("c")

    pltpu.sync_copy(ids_hbm.at[pl.ds(wid * ROWS_PER_W, ROWS_PER_W)], idx_v)

    pltpu.sync_copy(pos_hbm.at[pl.ds(0, S)], bias_v.at[pl.ds(0, S)])
    pltpu.sync_copy(pos_hbm.at[pl.ds(0, SEXT - S)], bias_v.at[pl.ds(S, SEXT - S)])
    pltpu.sync_copy(tt_hbm, tt_v)
    tt_vecs = [tt_v[0, pl.ds(h * LANES, LANES)] for h in range(NH)]

    @plsc.parallel_loop(0, SEXT)
    def bias_row(s):
        for h in range(NH):
            plsc.addupdate(bias_v.at[s, pl.ds(h * LANES, LANES)], tt_vecs[h])

    out_base = wid * ROWS_PER_W

    def gather_start(c, b):
        pltpu.async_copy(word_hbm.at[idx_v.at[pl.ds(c * CHUNK, CHUNK)]],
                         rows_v.at[b], gsem.at[b])

    def gather_wait(c, b):
        pltpu.make_async_copy(word_hbm.at[idx_v.at[pl.ds(c * CHUNK, CHUNK)]],
                              rows_v.at[b], gsem.at[b]).wait()

    def scatter_start(c, b):
        pltpu.async_copy(rows_v.at[b],
                         out_hbm.at[pl.ds(out_base + c * CHUNK, CHUNK)],
                         ssem.at[b])

    def scatter_wait(c, b):
        pltpu.make_async_copy(rows_v.at[b],
                              out_hbm.at[pl.ds(out_base + c * CHUNK, CHUNK)],
                              ssem.at[b]).wait()

    def add_bias(c, b):
        s0 = lax.rem(c * CHUNK, S)

        @plsc.parallel_loop(0, CHUNK, unroll=4)
        def add_row(i):
            for h in range(NH):
                sl = pl.ds(h * LANES, LANES)
                plsc.addupdate(rows_v.at[b, i, sl], bias_v[s0 + i, sl])

    def visit(c, b, swait, gstart):
        bn = (b + LOOKAHEAD) % NBUF
        if swait:
            scatter_wait(c - (NBUF - LOOKAHEAD), bn)
        if gstart:
            gather_start(c + LOOKAHEAD, bn)
        gather_wait(c, b)
        add_bias(c, b)
        scatter_start(c, b)

    for b in range(LOOKAHEAD):
        gather_start(b, b)

    for b in range(NBUF):
        visit(b, b, swait=(b >= NBUF - LOOKAHEAD), gstart=True)

    def round_body(r, carry):
        for b in range(NBUF):
            visit(r * NBUF + b, b, swait=True, gstart=True)
        return carry

    lax.fori_loop(1, ROUNDS - 1, round_body, None)

    c0 = (ROUNDS - 1) * NBUF
    for b in range(NBUF):
        last = c0 + b + LOOKAHEAD < CHUNKS_PER_W
        visit(c0 + b, b, swait=last, gstart=last)

    for b in range(NBUF):
        scatter_wait(c0 + b, b)


@jax.jit
def kernel(input_ids, word_embeddings, token_type_embeddings, position_embeddings):
    ids_flat = input_ids.astype(jnp.int32).reshape(TOTAL)

    mesh = plsc.VectorSubcoreMesh(
        core_axis_name="c", subcore_axis_name="s",
        num_cores=NUM_CORES, num_subcores=NUM_SUBCORES)

    run = pl.kernel(
        _sc_body,
        out_type=jax.ShapeDtypeStruct((TOTAL, HIDDEN), jnp.float32),
        mesh=mesh,
        scratch_types=[
            pltpu.VMEM((ROWS_PER_W,), jnp.int32),
            pltpu.VMEM((SEXT, HIDDEN), jnp.float32),
            pltpu.VMEM((2, HIDDEN), jnp.float32),
            pltpu.VMEM((NBUF, CHUNK, HIDDEN), jnp.float32),
            pltpu.SemaphoreType.DMA((NBUF,)),
            pltpu.SemaphoreType.DMA((NBUF,)),
        ],
    )
    out = run(ids_flat, word_embeddings, token_type_embeddings,
              position_embeddings)
    return out.reshape(B, S, HIDDEN)
